# baseline (speedup 1.0000x reference)
import functools

import jax
import jax.numpy as jnp
from jax import lax
from jax.experimental import pallas as pl
from jax.experimental.pallas import tpu as pltpu
from jax.experimental.pallas import tpu_sc as plsc

N_ENT = 10000
D = 128
EXT = 144
E = 320000
B = 4096
R2 = 200

_info = plsc.get_sparse_core_info()
NC = _info.num_cores
NS = _info.num_subcores
NW = NC * NS
EPW = E // NW
CHUNK = 50
NJ = EPW // CHUNK
JBLK = 20
NBLK = NJ // JBLK
N_PAD = 10240
RPT = N_PAD // NS

_mesh = plsc.VectorSubcoreMesh(core_axis_name="c", subcore_axis_name="s")


@functools.partial(
    pl.kernel,
    mesh=_mesh,
    compiler_params=pltpu.CompilerParams(use_tc_tiling_on_sc=False),
    out_type=jax.ShapeDtypeStruct((NC, N_PAD, EXT), jnp.float32),
    scratch_types=[
        pltpu.VMEM((2, JBLK, 3, CHUNK), jnp.int32),
        pltpu.VMEM((CHUNK, EXT), jnp.float32),
        pltpu.VMEM((CHUNK, EXT), jnp.float32),
        pltpu.VMEM((CHUNK, EXT), jnp.float32),
        pltpu.VMEM((CHUNK, EXT), jnp.float32),
        pltpu.VMEM_SHARED((N_PAD, EXT), jnp.float32),
        pltpu.VMEM_SHARED((R2, EXT), jnp.float32),
        pltpu.SemaphoreType.DMA,
        pltpu.SemaphoreType.DMA,
        pltpu.SemaphoreType.DMA,
        pltpu.SemaphoreType.DMA,
        pltpu.SemaphoreType.DMA,
        pltpu.SemaphoreType.DMA,
        pltpu.SemaphoreType.DMA,
        pltpu.SemaphoreType.DMA,
    ],
)
def _sc_aggregate(embed_ext_hbm, negrel_ext_hbm, idx_hbm, zeros_hbm, out_hbm,
                  idx_v, bufa0, bufa1, bufb0, bufb1, acc, negrel_sp,
                  sga0, sga1, sgb0, sgb1, ssa0, ssa1, ssb0, ssb1):
    cid = lax.axis_index("c")
    sid = lax.axis_index("s")
    wid = cid * NS + sid
    bufas = (bufa0, bufa1)
    bufbs = (bufb0, bufb1)
    sgas = (sga0, sga1)
    sgbs = (sgb0, sgb1)
    ssas = (ssa0, ssa1)
    ssbs = (ssb0, ssb1)
    pltpu.sync_copy(zeros_hbm, acc.at[pl.ds(sid * RPT, RPT)])

    @pl.when(sid == 0)
    def _stage_rel():
        pltpu.sync_copy(negrel_ext_hbm, negrel_sp)

    plsc.subcore_barrier()

    def block(t, carry):
        tb = lax.rem(t, 2)
        pltpu.sync_copy(idx_hbm.at[wid, pl.ds(t * JBLK, JBLK)], idx_v.at[tb])

        def pair(p, c2):
            nonfirst = jnp.logical_or(t > 0, p > 0)
            for s in range(2):
                j = 2 * p + s

                @pl.when(nonfirst)
                def _drain(s=s, j=j):
                    pltpu.make_async_copy(
                        bufas[s], acc.at[idx_v.at[tb, j, 1]], ssas[s]).wait()
                    pltpu.make_async_copy(
                        bufbs[s], acc.at[idx_v.at[tb, j, 1]], ssbs[s]).wait()

                pltpu.async_copy(embed_ext_hbm.at[idx_v.at[tb, j, 0]],
                                 bufas[s], sgas[s])
                pltpu.async_copy(negrel_sp.at[idx_v.at[tb, j, 2]],
                                 bufbs[s], sgbs[s])
            for s in range(2):
                j = 2 * p + s
                pltpu.make_async_copy(embed_ext_hbm.at[idx_v.at[tb, j, 0]],
                                      bufas[s], sgas[s]).wait()
                pltpu.async_copy(bufas[s], acc.at[idx_v.at[tb, j, 1]],
                                 ssas[s], add=True)
                pltpu.make_async_copy(negrel_sp.at[idx_v.at[tb, j, 2]],
                                      bufbs[s], sgbs[s]).wait()
                pltpu.async_copy(bufbs[s], acc.at[idx_v.at[tb, j, 1]],
                                 ssbs[s], add=True)
            return c2

        lax.fori_loop(0, JBLK // 2, pair, 0)
        return carry

    lax.fori_loop(0, NBLK, block, 0)
    for s in range(2):
        pltpu.make_async_copy(bufas[s], acc.at[idx_v.at[0, 0, 1]],
                              ssas[s]).wait()
        pltpu.make_async_copy(bufbs[s], acc.at[idx_v.at[0, 0, 1]],
                              ssbs[s]).wait()
    plsc.subcore_barrier()
    pltpu.sync_copy(acc.at[pl.ds(sid * RPT, RPT)],
                    out_hbm.at[cid, pl.ds(sid * RPT, RPT)])


BPW = B // NW


@functools.partial(
    pl.kernel,
    mesh=_mesh,
    compiler_params=pltpu.CompilerParams(use_tc_tiling_on_sc=False),
    out_type=(jax.ShapeDtypeStruct((B, D), jnp.float32),
              jax.ShapeDtypeStruct((B, D), jnp.float32)),
    scratch_types=[
        pltpu.VMEM((BPW,), jnp.int32),
        pltpu.VMEM((BPW,), jnp.int32),
        pltpu.VMEM((BPW, D), jnp.float32),
        pltpu.VMEM((BPW, D), jnp.float32),
        pltpu.SemaphoreType.DMA,
        pltpu.SemaphoreType.DMA,
    ],
)
def _sc_lookup(x_hbm, r_hbm, sub_hbm, rel_hbm, sub_out, rel_out,
               sub_v, rel_v, buf_x, buf_r, sem_x, sem_r):
    wid = lax.axis_index("c") * NS + lax.axis_index("s")
    base = wid * BPW
    pltpu.sync_copy(sub_hbm.at[pl.ds(base, BPW)], sub_v)
    pltpu.sync_copy(rel_hbm.at[pl.ds(base, BPW)], rel_v)
    cp_x = pltpu.async_copy(x_hbm.at[sub_v], buf_x, sem_x)
    cp_r = pltpu.async_copy(r_hbm.at[rel_v], buf_r, sem_r)
    cp_x.wait()
    cp_r.wait()
    pltpu.sync_copy(buf_x, sub_out.at[pl.ds(base, BPW)])
    pltpu.sync_copy(buf_r, rel_out.at[pl.ds(base, BPW)])


def _tc_dense(acc_ref, embed_ref, w_ref, wl_ref, b_ref, rel_ref, wr_ref,
              x_ref, r_ref):
    pre = acc_ref[0, :N_ENT, :D] + acc_ref[1, :N_ENT, :D]
    deg = acc_ref[0, :N_ENT, D:D + 1] + acc_ref[1, :N_ENT, D:D + 1]
    norm = 1.0 / jnp.maximum(deg, 1.0)
    agg = jnp.dot(pre, w_ref[...], preferred_element_type=jnp.float32) * norm
    loop = jnp.dot(embed_ref[...], wl_ref[...], preferred_element_type=jnp.float32)
    x_ref[...] = jnp.tanh(agg + loop + b_ref[...])
    r_ref[...] = jnp.dot(rel_ref[...], wr_ref[...], preferred_element_type=jnp.float32)


_tc_dense_call = pl.pallas_call(
    _tc_dense,
    out_shape=(jax.ShapeDtypeStruct((N_ENT, D), jnp.float32),
               jax.ShapeDtypeStruct((R2, D), jnp.float32)),
)


def kernel(init_embed, init_rel, W, W_loop, W_rel, b, edge_index, edge_type,
           sub, rel):
    f32 = jnp.float32
    eidx = edge_index.astype(jnp.int32).reshape(2, NW, NJ, CHUNK)
    typ = edge_type.astype(jnp.int32).reshape(NW, NJ, CHUNK)
    idx_packed = jnp.stack([eidx[0], eidx[1], typ], axis=2)
    embed_ext = jnp.concatenate(
        [init_embed.astype(f32),
         jnp.ones((N_ENT, 1), f32),
         jnp.zeros((N_ENT, EXT - D - 1), f32)], axis=1)
    negrel_ext = jnp.concatenate(
        [-init_rel.astype(f32), jnp.zeros((R2, EXT - D), f32)], axis=1)
    zeros_blk = jnp.zeros((RPT, EXT), f32)

    acc2 = _sc_aggregate(embed_ext, negrel_ext, idx_packed, zeros_blk)
    x, r = _tc_dense_call(acc2, init_embed, W, W_loop,
                          b.reshape(1, D), init_rel, W_rel)
    sub_emb, rel_emb = _sc_lookup(x, r, sub.astype(jnp.int32),
                                  rel.astype(jnp.int32))
    return (sub_emb, rel_emb, x)


# speedup vs baseline: 1.1481x; 1.1481x over previous
import functools

import jax
import jax.numpy as jnp
from jax import lax
from jax.experimental import pallas as pl
from jax.experimental.pallas import tpu as pltpu
from jax.experimental.pallas import tpu_sc as plsc

N_ENT = 10000
D = 128
EXT = 144
E = 320000
B = 4096
R2 = 200

_info = plsc.get_sparse_core_info()
NC = _info.num_cores
NS = _info.num_subcores
NW = NC * NS
EPW = E // NW
CHUNK = 100
NJ = EPW // CHUNK
JBLK = 10
NBLK = NJ // JBLK
N_PAD = 10240
RPT = N_PAD // NS

_mesh = plsc.VectorSubcoreMesh(core_axis_name="c", subcore_axis_name="s")


@functools.partial(
    pl.kernel,
    mesh=_mesh,
    compiler_params=pltpu.CompilerParams(use_tc_tiling_on_sc=False),
    out_type=(jax.ShapeDtypeStruct((NC, N_PAD, EXT), jnp.float32),
              jax.ShapeDtypeStruct((NC, B, EXT), jnp.float32),
              jax.ShapeDtypeStruct((NC, B, EXT), jnp.float32)),
    scratch_types=[
        pltpu.VMEM((2, JBLK, 3, CHUNK), jnp.int32),
        pltpu.VMEM((CHUNK, EXT), jnp.float32),
        pltpu.VMEM((CHUNK, EXT), jnp.float32),
        pltpu.VMEM_SHARED((N_PAD, EXT), jnp.float32),
        pltpu.VMEM_SHARED((R2, EXT), jnp.float32),
        pltpu.SemaphoreType.DMA,
        pltpu.SemaphoreType.DMA,
        pltpu.SemaphoreType.DMA,
        pltpu.SemaphoreType.DMA,
        pltpu.SemaphoreType.DMA,
        pltpu.SemaphoreType.DMA,
    ],
)
def _sc_aggregate(embed_ext_hbm, negrel_ext_hbm, idx_hbm, zeros_hbm,
                  sub_hbm, rel_hbm, out_hbm, subpre_out, gath_out,
                  idx_v, buf0, buf1, acc, negrel_sp,
                  sga0, sga1, sgb0, sgb1, ssc0, ssc1):
    cid = lax.axis_index("c")
    sid = lax.axis_index("s")
    wid = cid * NS + sid
    bufs = (buf0, buf1)
    sgas = (sga0, sga1)
    sgbs = (sgb0, sgb1)
    sscs = (ssc0, ssc1)
    pltpu.sync_copy(zeros_hbm, acc.at[pl.ds(sid * RPT, RPT)])

    @pl.when(sid == 0)
    def _stage_rel():
        pltpu.sync_copy(negrel_ext_hbm, negrel_sp)

    plsc.subcore_barrier()

    def block(t, carry):
        tb = lax.rem(t, 2)
        pltpu.sync_copy(idx_hbm.at[wid, pl.ds(t * JBLK, JBLK)], idx_v.at[tb])

        def pair(p, c2):
            nonfirst = jnp.logical_or(t > 0, p > 0)
            for s in range(2):
                j = 2 * p + s

                @pl.when(nonfirst)
                def _drain(s=s, j=j):
                    pltpu.make_async_copy(
                        bufs[s], acc.at[idx_v.at[tb, j, 1]], sscs[s]).wait()

                pltpu.async_copy(embed_ext_hbm.at[idx_v.at[tb, j, 0]],
                                 bufs[s], sgas[s])
            for s in range(2):
                j = 2 * p + s
                pltpu.make_async_copy(embed_ext_hbm.at[idx_v.at[tb, j, 0]],
                                      bufs[s], sgas[s]).wait()
                pltpu.async_copy(negrel_sp.at[idx_v.at[tb, j, 2]],
                                 bufs[s], sgbs[s], add=True)
            for s in range(2):
                j = 2 * p + s
                pltpu.make_async_copy(negrel_sp.at[idx_v.at[tb, j, 2]],
                                      bufs[s], sgbs[s]).wait()
                pltpu.async_copy(bufs[s], acc.at[idx_v.at[tb, j, 1]],
                                 sscs[s], add=True)
            return c2

        lax.fori_loop(0, JBLK // 2, pair, 0)
        return carry

    lax.fori_loop(0, NBLK, block, 0)
    for s in range(2):
        pltpu.make_async_copy(bufs[s], acc.at[idx_v.at[0, 0, 1]],
                              sscs[s]).wait()
    plsc.subcore_barrier()
    pltpu.sync_copy(acc.at[pl.ds(sid * RPT, RPT)],
                    out_hbm.at[cid, pl.ds(sid * RPT, RPT)])

    idx_s = idx_v.at[0, 0, 0]

    def _gather_rows(table, out_ref, out_c, list_hbm):
        for off, sz in ((0, 96), (96, 96), (192, 64)):
            base = sid * (B // NS) + off
            pltpu.sync_copy(list_hbm.at[pl.ds(base, sz)],
                            idx_s.at[pl.ds(0, sz)])
            pltpu.async_copy(table.at[idx_s.at[pl.ds(0, sz)]],
                             buf0.at[pl.ds(0, sz)], sga0).wait()
            pltpu.sync_copy(buf0.at[pl.ds(0, sz)],
                            out_ref.at[out_c, pl.ds(base, sz)])

    _gather_rows(acc, subpre_out, cid, sub_hbm)

    @pl.when(cid == 0)
    def _gather_sub_embed():
        _gather_rows(embed_ext_hbm, gath_out, 0, sub_hbm)

    @pl.when(cid == 1)
    def _gather_rel_rows():
        _gather_rows(negrel_sp, gath_out, 1, rel_hbm)


def _tc_dense(acc_ref, embed_ref, w_ref, wl_ref, b_ref, wr_ref,
              subpre_ref, gath_ref,
              x_ref, sub_ref, rel_ref):
    w = w_ref[...]
    wl = wl_ref[...]
    bb = b_ref[...]
    pre = acc_ref[0, :N_ENT, :D] + acc_ref[1, :N_ENT, :D]
    deg = acc_ref[0, :N_ENT, D:D + 1] + acc_ref[1, :N_ENT, D:D + 1]
    norm = 1.0 / jnp.maximum(deg, 1.0)
    agg = jnp.dot(pre, w, preferred_element_type=jnp.float32) * norm
    loop = jnp.dot(embed_ref[...], wl, preferred_element_type=jnp.float32)
    x_ref[...] = jnp.tanh(agg + loop + bb)

    sp = subpre_ref[0, :, :D] + subpre_ref[1, :, :D]
    sdeg = subpre_ref[0, :, D:D + 1] + subpre_ref[1, :, D:D + 1]
    snorm = 1.0 / jnp.maximum(sdeg, 1.0)
    sagg = jnp.dot(sp, w, preferred_element_type=jnp.float32) * snorm
    sloop = jnp.dot(gath_ref[0, :, :D], wl, preferred_element_type=jnp.float32)
    sub_ref[...] = jnp.tanh(sagg + sloop + bb)

    rel_ref[...] = jnp.dot(-gath_ref[1, :, :D], wr_ref[...],
                           preferred_element_type=jnp.float32)


_tc_dense_call = pl.pallas_call(
    _tc_dense,
    out_shape=(jax.ShapeDtypeStruct((N_ENT, D), jnp.float32),
               jax.ShapeDtypeStruct((B, D), jnp.float32),
               jax.ShapeDtypeStruct((B, D), jnp.float32)),
)


def kernel(init_embed, init_rel, W, W_loop, W_rel, b, edge_index, edge_type,
           sub, rel):
    f32 = jnp.float32
    eidx = edge_index.astype(jnp.int32).reshape(2, NW, NJ, CHUNK)
    typ = edge_type.astype(jnp.int32).reshape(NW, NJ, CHUNK)
    idx_packed = jnp.stack([eidx[0], eidx[1], typ], axis=2)
    embed_ext = jnp.concatenate(
        [init_embed.astype(f32),
         jnp.ones((N_ENT, 1), f32),
         jnp.zeros((N_ENT, EXT - D - 1), f32)], axis=1)
    negrel_ext = jnp.concatenate(
        [-init_rel.astype(f32), jnp.zeros((R2, EXT - D), f32)], axis=1)
    zeros_blk = jnp.zeros((RPT, EXT), f32)

    acc2, subpre2, gath2 = _sc_aggregate(
        embed_ext, negrel_ext, idx_packed, zeros_blk,
        sub.astype(jnp.int32), rel.astype(jnp.int32))
    x, sub_emb, rel_emb = _tc_dense_call(
        acc2, init_embed, W, W_loop, b.reshape(1, D), W_rel, subpre2, gath2)
    return (sub_emb, rel_emb, x)


# speedup vs baseline: 1.2128x; 1.0563x over previous
import functools

import jax
import jax.numpy as jnp
from jax import lax
from jax.experimental import pallas as pl
from jax.experimental.pallas import tpu as pltpu
from jax.experimental.pallas import tpu_sc as plsc

N_ENT = 10000
D = 128
EXT = 144
E = 320000
B = 4096
R2 = 200

_info = plsc.get_sparse_core_info()
NC = _info.num_cores
NS = _info.num_subcores
NW = NC * NS
EPW = E // NW
CHUNK = 100
NJ = EPW // CHUNK
JBLK = 10
NBLK = NJ // JBLK
N_PAD = 10240
RPT = N_PAD // NS

_mesh = plsc.VectorSubcoreMesh(core_axis_name="c", subcore_axis_name="s")


@functools.partial(
    pl.kernel,
    mesh=_mesh,
    compiler_params=pltpu.CompilerParams(use_tc_tiling_on_sc=False),
    out_type=jax.ShapeDtypeStruct((NC, N_PAD, EXT), jnp.float32),
    scratch_types=[
        pltpu.VMEM((2, JBLK, 3, CHUNK), jnp.int32),
        pltpu.VMEM((CHUNK, EXT), jnp.float32),
        pltpu.VMEM((CHUNK, EXT), jnp.float32),
        pltpu.VMEM_SHARED((N_PAD, EXT), jnp.float32),
        pltpu.VMEM_SHARED((R2, EXT), jnp.float32),
        pltpu.SemaphoreType.DMA,
        pltpu.SemaphoreType.DMA,
        pltpu.SemaphoreType.DMA,
        pltpu.SemaphoreType.DMA,
        pltpu.SemaphoreType.DMA,
        pltpu.SemaphoreType.DMA,
    ],
)
def _sc_aggregate(embed_ext_hbm, negrel_ext_hbm, idx_hbm, zeros_hbm, out_hbm,
                  idx_v, buf0, buf1, acc, negrel_sp,
                  sga0, sga1, sgb0, sgb1, ssc0, ssc1):
    cid = lax.axis_index("c")
    sid = lax.axis_index("s")
    wid = cid * NS + sid
    bufs = (buf0, buf1)
    sgas = (sga0, sga1)
    sgbs = (sgb0, sgb1)
    sscs = (ssc0, ssc1)
    pltpu.sync_copy(zeros_hbm, acc.at[pl.ds(sid * RPT, RPT)])

    @pl.when(sid == 0)
    def _stage_rel():
        pltpu.sync_copy(negrel_ext_hbm, negrel_sp)

    plsc.subcore_barrier()

    def block(t, carry):
        tb = lax.rem(t, 2)
        pltpu.sync_copy(idx_hbm.at[wid, pl.ds(t * JBLK, JBLK)], idx_v.at[tb])

        def pair(p, c2):
            nonfirst = jnp.logical_or(t > 0, p > 0)
            for s in range(2):
                j = 2 * p + s

                @pl.when(nonfirst)
                def _drain(s=s, j=j):
                    pltpu.make_async_copy(
                        bufs[s], acc.at[idx_v.at[tb, j, 1]], sscs[s]).wait()

                pltpu.async_copy(embed_ext_hbm.at[idx_v.at[tb, j, 0]],
                                 bufs[s], sgas[s])
            for s in range(2):
                j = 2 * p + s
                pltpu.make_async_copy(embed_ext_hbm.at[idx_v.at[tb, j, 0]],
                                      bufs[s], sgas[s]).wait()
                pltpu.async_copy(negrel_sp.at[idx_v.at[tb, j, 2]],
                                 bufs[s], sgbs[s], add=True)
            for s in range(2):
                j = 2 * p + s
                pltpu.make_async_copy(negrel_sp.at[idx_v.at[tb, j, 2]],
                                      bufs[s], sgbs[s]).wait()
                pltpu.async_copy(bufs[s], acc.at[idx_v.at[tb, j, 1]],
                                 sscs[s], add=True)
            return c2

        lax.fori_loop(0, JBLK // 2, pair, 0)
        return carry

    lax.fori_loop(0, NBLK, block, 0)
    for s in range(2):
        pltpu.make_async_copy(bufs[s], acc.at[idx_v.at[0, 0, 1]],
                              sscs[s]).wait()
    plsc.subcore_barrier()
    pltpu.sync_copy(acc.at[pl.ds(sid * RPT, RPT)],
                    out_hbm.at[cid, pl.ds(sid * RPT, RPT)])


BPW = B // NW


@functools.partial(
    pl.kernel,
    mesh=_mesh,
    compiler_params=pltpu.CompilerParams(use_tc_tiling_on_sc=False),
    out_type=(jax.ShapeDtypeStruct((B, D), jnp.float32),
              jax.ShapeDtypeStruct((B, D), jnp.float32)),
    scratch_types=[
        pltpu.VMEM((BPW,), jnp.int32),
        pltpu.VMEM((BPW,), jnp.int32),
        pltpu.VMEM((BPW, D), jnp.float32),
        pltpu.VMEM((BPW, D), jnp.float32),
        pltpu.SemaphoreType.DMA,
        pltpu.SemaphoreType.DMA,
    ],
)
def _sc_lookup(x_hbm, r_hbm, sub_hbm, rel_hbm, sub_out, rel_out,
               sub_v, rel_v, buf_x, buf_r, sem_x, sem_r):
    wid = lax.axis_index("c") * NS + lax.axis_index("s")
    base = wid * BPW
    pltpu.sync_copy(sub_hbm.at[pl.ds(base, BPW)], sub_v)
    pltpu.sync_copy(rel_hbm.at[pl.ds(base, BPW)], rel_v)
    cp_x = pltpu.async_copy(x_hbm.at[sub_v], buf_x, sem_x)
    cp_r = pltpu.async_copy(r_hbm.at[rel_v], buf_r, sem_r)
    cp_x.wait()
    cp_r.wait()
    pltpu.sync_copy(buf_x, sub_out.at[pl.ds(base, BPW)])
    pltpu.sync_copy(buf_r, rel_out.at[pl.ds(base, BPW)])


def _tc_dense(acc_ref, embed_ref, w_ref, wl_ref, b_ref, rel_ref, wr_ref,
              x_ref, r_ref):
    pre = acc_ref[0, :N_ENT, :D] + acc_ref[1, :N_ENT, :D]
    deg = acc_ref[0, :N_ENT, D:D + 1] + acc_ref[1, :N_ENT, D:D + 1]
    norm = 1.0 / jnp.maximum(deg, 1.0)
    agg = jnp.dot(pre, w_ref[...], preferred_element_type=jnp.float32) * norm
    loop = jnp.dot(embed_ref[...], wl_ref[...], preferred_element_type=jnp.float32)
    x_ref[...] = jnp.tanh(agg + loop + b_ref[...])
    r_ref[...] = jnp.dot(rel_ref[...], wr_ref[...], preferred_element_type=jnp.float32)


_tc_dense_call = pl.pallas_call(
    _tc_dense,
    out_shape=(jax.ShapeDtypeStruct((N_ENT, D), jnp.float32),
               jax.ShapeDtypeStruct((R2, D), jnp.float32)),
)


def kernel(init_embed, init_rel, W, W_loop, W_rel, b, edge_index, edge_type,
           sub, rel):
    f32 = jnp.float32
    eidx = edge_index.astype(jnp.int32).reshape(2, NW, NJ, CHUNK)
    typ = edge_type.astype(jnp.int32).reshape(NW, NJ, CHUNK)
    idx_packed = jnp.stack([eidx[0], eidx[1], typ], axis=2)
    embed_ext = jnp.concatenate(
        [init_embed.astype(f32),
         jnp.ones((N_ENT, 1), f32),
         jnp.zeros((N_ENT, EXT - D - 1), f32)], axis=1)
    negrel_ext = jnp.concatenate(
        [-init_rel.astype(f32), jnp.zeros((R2, EXT - D), f32)], axis=1)
    zeros_blk = jnp.zeros((RPT, EXT), f32)

    acc2 = _sc_aggregate(embed_ext, negrel_ext, idx_packed, zeros_blk)
    x, r = _tc_dense_call(acc2, init_embed, W, W_loop,
                          b.reshape(1, D), init_rel, W_rel)
    sub_emb, rel_emb = _sc_lookup(x, r, sub.astype(jnp.int32),
                                  rel.astype(jnp.int32))
    return (sub_emb, rel_emb, x)
